# baseline (speedup 1.0000x reference)
import jax
import jax.numpy as jnp
from jax import lax
from jax.experimental import pallas as pl
from jax.experimental.pallas import tpu as pltpu
from jax.experimental.pallas import tpu_sc as plsc

NC = 2
NS = 16
NW = NC * NS
LANES = 16

B, L, D = 4096, 200, 64
FLAT = B * L
PER_W = FLAT // NW
SUB = 100
SPC = 2
CHUNK = SUB * SPC
NCHUNK = PER_W // CHUNK


def _sc_body(table_hbm, idx_hbm, pos_hbm, out_hbm,
             idx_v, pos_v, rows_v, sem_g0, sem_g1, sem_w0, sem_w1):
    wid = lax.axis_index("s") * NC + lax.axis_index("c")
    pltpu.sync_copy(idx_hbm.at[wid], idx_v)
    pltpu.sync_copy(pos_hbm, pos_v)

    out_base = wid * NCHUNK
    sem_g = (sem_g0, sem_g1)
    sem_w = (sem_w0, sem_w1)

    def gather(k, b, op):
        for j in range(SPC):
            cp = pltpu.make_async_copy(
                table_hbm.at[idx_v.at[k * SPC + j]],
                rows_v.at[b, pl.ds(j * SUB, SUB)], sem_g[b])
            getattr(cp, op)()

    def write(k, b, op):
        cp = pltpu.make_async_copy(
            rows_v.at[b], out_hbm.at[out_base + k], sem_w[b])
        getattr(cp, op)()

    def add_pos(b):
        @pl.loop(0, CHUNK)
        def _row(r):
            for k in range(D // LANES):
                plsc.addupdate(
                    rows_v.at[b, r, pl.ds(k * LANES, LANES)],
                    pos_v[r, pl.ds(k * LANES, LANES)])

    gather(0, 0, "start")

    @pl.loop(0, NCHUNK, step=2)
    def _pair(c0):
        @pl.when(c0 >= 1)
        def _():
            write(c0, 1, "wait")
        gather(c0 + 1, 1, "start")
        gather(c0, 0, "wait")
        add_pos(0)
        write(c0, 0, "start")
        write(c0, 0, "wait")
        @pl.when(c0 < NCHUNK - 2)
        def _():
            gather(c0 + 2, 0, "start")
        gather(c0 + 1, 1, "wait")
        add_pos(1)
        write(c0 + 1, 1, "start")

    write(0, 1, "wait")


_sc_gather_add = pl.kernel(
    _sc_body,
    out_type=jax.ShapeDtypeStruct((B, L, D), jnp.float32),
    mesh=plsc.VectorSubcoreMesh(core_axis_name="c", subcore_axis_name="s"),
    scratch_types=[
        pltpu.VMEM((NCHUNK * SPC, SUB), jnp.int32),
        pltpu.VMEM((CHUNK, D), jnp.float32),
        pltpu.VMEM((2, CHUNK, D), jnp.float32),
        pltpu.SemaphoreType.DMA,
        pltpu.SemaphoreType.DMA,
        pltpu.SemaphoreType.DMA,
        pltpu.SemaphoreType.DMA,
    ],
    compiler_params=pltpu.CompilerParams(use_tc_tiling_on_sc=False),
)


@jax.jit
def kernel(x, item_table, pos_table):
    idx = x.reshape(NW, NCHUNK * SPC, SUB)
    pos = pos_table[1:L + 1]
    return _sc_gather_add(item_table, idx, pos)


# speedup vs baseline: 1.3152x; 1.3152x over previous
import jax
import jax.numpy as jnp
from jax import lax
from jax.experimental import pallas as pl
from jax.experimental.pallas import tpu as pltpu
from jax.experimental.pallas import tpu_sc as plsc

NC = 2
NS = 16
NW = NC * NS
LANES = 16

B, L, D = 4096, 200, 64
FLAT = B * L
PER_W = FLAT // NW
SUB = 100
SPC = 2
CHUNK = SUB * SPC
NCHUNK = PER_W // CHUNK


def _sc_body(table_hbm, idx_hbm, pos_hbm, out_hbm,
             idx_v, pos_v, rows_v, sem_g0, sem_g1, sem_w0, sem_w1):
    wid = lax.axis_index("s") * NC + lax.axis_index("c")
    pltpu.sync_copy(idx_hbm.at[wid], idx_v)
    pltpu.sync_copy(pos_hbm, pos_v)

    out_base = wid * NCHUNK
    sem_g = (sem_g0, sem_g1)
    sem_w = (sem_w0, sem_w1)

    def gather(k, b, op):
        for j in range(SPC):
            cp = pltpu.make_async_copy(
                table_hbm.at[idx_v.at[k * SPC + j]],
                rows_v.at[b, pl.ds(j * SUB, SUB)], sem_g[b])
            getattr(cp, op)()

    def write(k, b, op):
        cp = pltpu.make_async_copy(
            rows_v.at[b], out_hbm.at[out_base + k, :, pl.ds(0, D)], sem_w[b])
        getattr(cp, op)()

    def add_pos(b):
        @pl.loop(0, CHUNK)
        def _row(r):
            for k in range(D // LANES):
                plsc.addupdate(
                    rows_v.at[b, r, pl.ds(k * LANES, LANES)],
                    pos_v[r, pl.ds(k * LANES, LANES)])

    gather(0, 0, "start")

    @pl.loop(0, NCHUNK, step=2)
    def _pair(c0):
        @pl.when(c0 >= 1)
        def _():
            write(c0, 1, "wait")
        gather(c0 + 1, 1, "start")
        gather(c0, 0, "wait")
        add_pos(0)
        write(c0, 0, "start")
        write(c0, 0, "wait")
        @pl.when(c0 < NCHUNK - 2)
        def _():
            gather(c0 + 2, 0, "start")
        gather(c0 + 1, 1, "wait")
        add_pos(1)
        write(c0 + 1, 1, "start")

    write(0, 1, "wait")


_sc_gather_add = pl.kernel(
    _sc_body,
    out_type=jax.ShapeDtypeStruct((B, L, 2 * D), jnp.float32),
    mesh=plsc.VectorSubcoreMesh(core_axis_name="c", subcore_axis_name="s"),
    scratch_types=[
        pltpu.VMEM((NCHUNK * SPC, SUB), jnp.int32),
        pltpu.VMEM((CHUNK, D), jnp.float32),
        pltpu.VMEM((2, CHUNK, D), jnp.float32),
        pltpu.SemaphoreType.DMA,
        pltpu.SemaphoreType.DMA,
        pltpu.SemaphoreType.DMA,
        pltpu.SemaphoreType.DMA,
    ],
    compiler_params=pltpu.CompilerParams(use_tc_tiling_on_sc=False),
)


@jax.jit
def kernel(x, item_table, pos_table):
    idx = x.reshape(NW, NCHUNK * SPC, SUB)
    pos = pos_table[1:L + 1]
    out5 = _sc_gather_add(item_table, idx, pos)
    return out5[:, :, :D]


# speedup vs baseline: 1.3546x; 1.0300x over previous
import jax
import jax.numpy as jnp
from jax import lax
from jax.experimental import pallas as pl
from jax.experimental.pallas import tpu as pltpu
from jax.experimental.pallas import tpu_sc as plsc

NC = 2
NS = 16
NW = NC * NS
LANES = 16
NBUF = 3

B, L, D = 4096, 200, 64
FLAT = B * L
PER_W = FLAT // NW
SUB = 100
SPC = 2
CHUNK = SUB * SPC
NCHUNK = PER_W // CHUNK
NMAIN = (NCHUNK // NBUF) * NBUF


def _sc_body(table_hbm, idx_hbm, pos_hbm, out_hbm,
             idx_v, pos_v, rows_v, *sems):
    wid = lax.axis_index("s") * NC + lax.axis_index("c")
    pltpu.sync_copy(idx_hbm.at[wid], idx_v)
    pltpu.sync_copy(pos_hbm, pos_v)

    out_base = wid * NCHUNK
    sem_g = sems[:NBUF]
    sem_w = sems[NBUF:]

    def gather(k, b, op):
        for j in range(SPC):
            cp = pltpu.make_async_copy(
                table_hbm.at[idx_v.at[k * SPC + j]],
                rows_v.at[b, pl.ds(j * SUB, SUB)], sem_g[b])
            getattr(cp, op)()

    def write(k, b, op):
        cp = pltpu.make_async_copy(
            rows_v.at[b], out_hbm.at[out_base + k, :, pl.ds(0, D)], sem_w[b])
        getattr(cp, op)()

    def add_pos(b):
        @pl.loop(0, CHUNK)
        def _row(r):
            for k in range(D // LANES):
                plsc.addupdate(
                    rows_v.at[b, r, pl.ds(k * LANES, LANES)],
                    pos_v[r, pl.ds(k * LANES, LANES)])

    gather(0, 0, "start")

    @pl.loop(0, NMAIN, step=NBUF)
    def _ring(c0):
        for i in range(NBUF):
            k = c0 + i
            b = i
            @pl.when(k >= 2)
            def _(k=k, b=b):
                write(k, (b + 1) % NBUF, "wait")
            @pl.when(k < NCHUNK - 1)
            def _(k=k, b=b):
                gather(k + 1, (b + 1) % NBUF, "start")
            gather(k, b, "wait")
            add_pos(b)
            write(k, b, "start")

    for k in range(NMAIN, NCHUNK):
        b = k % NBUF
        write(k, (b + 1) % NBUF, "wait")
        if k + 1 < NCHUNK:
            gather(k + 1, (b + 1) % NBUF, "start")
        gather(k, b, "wait")
        add_pos(b)
        write(k, b, "start")

    for k in range(NCHUNK - NBUF + 1, NCHUNK):
        write(k, k % NBUF, "wait")


_sc_gather_add = pl.kernel(
    _sc_body,
    out_type=jax.ShapeDtypeStruct((B, L, 2 * D), jnp.float32),
    mesh=plsc.VectorSubcoreMesh(core_axis_name="c", subcore_axis_name="s"),
    scratch_types=[
        pltpu.VMEM((NCHUNK * SPC, SUB), jnp.int32),
        pltpu.VMEM((CHUNK, D), jnp.float32),
        pltpu.VMEM((NBUF, CHUNK, D), jnp.float32),
    ] + [pltpu.SemaphoreType.DMA] * (2 * NBUF),
    compiler_params=pltpu.CompilerParams(use_tc_tiling_on_sc=False),
)


@jax.jit
def kernel(x, item_table, pos_table):
    idx = x.reshape(NW, NCHUNK * SPC, SUB)
    pos = pos_table[1:L + 1]
    out5 = _sc_gather_add(item_table, idx, pos)
    return out5[:, :, :D]


# speedup vs baseline: 1.3742x; 1.0145x over previous
import jax
import jax.numpy as jnp
from jax import lax
from jax.experimental import pallas as pl
from jax.experimental.pallas import tpu as pltpu
from jax.experimental.pallas import tpu_sc as plsc

NC = 2
NS = 16
NW = NC * NS
LANES = 16
NBUF = 4

B, L, D = 4096, 200, 64
FLAT = B * L
PER_W = FLAT // NW
SUB = 100
SPC = 2
CHUNK = SUB * SPC
NCHUNK = PER_W // CHUNK
NMAIN = (NCHUNK // NBUF) * NBUF


def _sc_body(table_hbm, idx_hbm, pos_hbm, out_hbm,
             idx_v, pos_v, rows_v, *sems):
    wid = lax.axis_index("s") * NC + lax.axis_index("c")
    pltpu.sync_copy(idx_hbm.at[wid], idx_v)
    pltpu.sync_copy(pos_hbm, pos_v)

    out_base = wid * NCHUNK
    sem_g = sems[:NBUF]
    sem_w = sems[NBUF:]

    def gather(k, b, op):
        for j in range(SPC):
            cp = pltpu.make_async_copy(
                table_hbm.at[idx_v.at[k * SPC + j]],
                rows_v.at[b, pl.ds(j * SUB, SUB)], sem_g[b])
            getattr(cp, op)()

    def write(k, b, op):
        cp = pltpu.make_async_copy(
            rows_v.at[b], out_hbm.at[out_base + k, :, pl.ds(0, D)], sem_w[b])
        getattr(cp, op)()

    def add_pos(b):
        @pl.loop(0, CHUNK)
        def _row(r):
            for k in range(D // LANES):
                plsc.addupdate(
                    rows_v.at[b, r, pl.ds(k * LANES, LANES)],
                    pos_v[r, pl.ds(k * LANES, LANES)])

    gather(0, 0, "start")
    gather(1, 1, "start")

    @pl.loop(0, NMAIN, step=NBUF)
    def _ring(c0):
        for i in range(NBUF):
            k = c0 + i
            b = i
            @pl.when(k >= 2)
            def _(k=k, b=b):
                write(k, (b + 2) % NBUF, "wait")
            @pl.when(k < NCHUNK - 2)
            def _(k=k, b=b):
                gather(k + 2, (b + 2) % NBUF, "start")
            gather(k, b, "wait")
            add_pos(b)
            write(k, b, "start")

    for k in range(NCHUNK - 2, NCHUNK):
        write(k, k % NBUF, "wait")


_sc_gather_add = pl.kernel(
    _sc_body,
    out_type=jax.ShapeDtypeStruct((B, L, 2 * D), jnp.float32),
    mesh=plsc.VectorSubcoreMesh(core_axis_name="c", subcore_axis_name="s"),
    scratch_types=[
        pltpu.VMEM((NCHUNK * SPC, SUB), jnp.int32),
        pltpu.VMEM((CHUNK, D), jnp.float32),
        pltpu.VMEM((NBUF, CHUNK, D), jnp.float32),
    ] + [pltpu.SemaphoreType.DMA] * (2 * NBUF),
    compiler_params=pltpu.CompilerParams(use_tc_tiling_on_sc=False),
)


@jax.jit
def kernel(x, item_table, pos_table):
    idx = x.reshape(NW, NCHUNK * SPC, SUB)
    pos = pos_table[1:L + 1]
    out5 = _sc_gather_add(item_table, idx, pos)
    return out5[:, :, :D]
